# baseline (speedup 1.0000x reference)
import functools
import jax
import jax.numpy as jnp
from jax import lax
from jax.experimental import pallas as pl
from jax.experimental.pallas import tpu as pltpu
from jax.experimental.pallas import tpu_sc as plsc

N_PAD = 10240
ROW_BLK = 1024
D = 512
NC = 2
NS = 16
NW = NC * NS
CHUNKS = 2
N_RT = NW * CHUNKS
CH = N_PAD // N_RT
EB = 64
EPAD = 160000 + N_RT * EB


def _mm_a_body(x_ref, w_ref, dinv_ref, o_ref):
    o_ref[...] = jnp.dot(x_ref[...], w_ref[...],
                         preferred_element_type=jnp.float32) * dinv_ref[...]


def _mm_b_body(g_ref, dinv_ref, b_ref, w_ref, y_ref, h_ref):
    y = jnp.maximum(g_ref[...] * dinv_ref[...] + b_ref[...], 0.0)
    y_ref[...] = y
    h_ref[...] = jnp.dot(y, w_ref[...],
                         preferred_element_type=jnp.float32) * dinv_ref[...]


def _mm_br_body(g_ref, dinv_ref, b_ref, r_ref, w_ref, y_ref, h_ref):
    y = jnp.maximum(r_ref[...] + g_ref[...] * dinv_ref[...] + b_ref[...], 0.0)
    y_ref[...] = y
    h_ref[...] = jnp.dot(y, w_ref[...],
                         preferred_element_type=jnp.float32) * dinv_ref[...]


def _ew_body(g_ref, dinv_ref, b_ref, r_ref, y_ref):
    y_ref[...] = jnp.maximum(
        r_ref[...] + g_ref[...] * dinv_ref[...] + b_ref[...], 0.0)


def _row_spec(width):
    return pl.BlockSpec((ROW_BLK, width), lambda i: (i, 0))


def _full_spec(shape):
    return pl.BlockSpec(shape, lambda i: (0, 0))


def _mm_a(x, w, dinv):
    k = x.shape[1]
    n = w.shape[1]
    return pl.pallas_call(
        _mm_a_body,
        grid=(N_PAD // ROW_BLK,),
        in_specs=[_row_spec(k), _full_spec((k, n)), _row_spec(1)],
        out_specs=_row_spec(n),
        out_shape=jax.ShapeDtypeStruct((N_PAD, n), jnp.float32),
    )(x, w, dinv)


def _mm_b(g, dinv, b, w):
    n = w.shape[1]
    k = g.shape[1]
    return pl.pallas_call(
        _mm_b_body,
        grid=(N_PAD // ROW_BLK,),
        in_specs=[_row_spec(k), _row_spec(1), _full_spec((1, k)),
                  _full_spec((k, n))],
        out_specs=[_row_spec(k), _row_spec(n)],
        out_shape=[jax.ShapeDtypeStruct((N_PAD, k), jnp.float32),
                   jax.ShapeDtypeStruct((N_PAD, n), jnp.float32)],
    )(g, dinv, b.reshape(1, k), w)


def _mm_br(g, dinv, b, resid, w):
    n = w.shape[1]
    k = g.shape[1]
    return pl.pallas_call(
        _mm_br_body,
        grid=(N_PAD // ROW_BLK,),
        in_specs=[_row_spec(k), _row_spec(1), _full_spec((1, k)),
                  _row_spec(k), _full_spec((k, n))],
        out_specs=[_row_spec(k), _row_spec(n)],
        out_shape=[jax.ShapeDtypeStruct((N_PAD, k), jnp.float32),
                   jax.ShapeDtypeStruct((N_PAD, n), jnp.float32)],
    )(g, dinv, b.reshape(1, k), resid, w)


def _ew(g, dinv, b, resid):
    k = g.shape[1]
    return pl.pallas_call(
        _ew_body,
        grid=(N_PAD // ROW_BLK,),
        in_specs=[_row_spec(k), _row_spec(1), _full_spec((1, k)),
                  _row_spec(k)],
        out_specs=_row_spec(k),
        out_shape=jax.ShapeDtypeStruct((N_PAD, k), jnp.float32),
    )(g, dinv, b.reshape(1, k), resid)


def _sc_prop_body(hp_hbm, hp1d_hbm, srcp_hbm, dstl_hbm, desc_hbm, out_hbm,
                  desc_v, idx_v, dst_v, rows_v, acc_v, gsem):
    c = lax.axis_index("c")
    s = lax.axis_index("s")
    wid = c * NS + s
    pltpu.sync_copy(desc_hbm.at[wid], desc_v)
    dv = desc_v[...]
    iota = lax.iota(jnp.int32, 16)
    for k in range(CHUNKS):
        rbase = (wid * CHUNKS + k) * CH
        pltpu.sync_copy(hp1d_hbm.at[pl.ds(rbase * D, CH * D)],
                        acc_v.at[pl.ds(0, CH * D)])
        start = dv[2 * k]
        nb = dv[2 * k + 1]

        def body(i, carry):
            slot = pl.multiple_of((start + i) * EB, 8)
            pltpu.sync_copy(srcp_hbm.at[pl.ds(slot, EB)], idx_v)
            pltpu.sync_copy(dstl_hbm.at[pl.ds(slot, EB)],
                            dst_v.at[pl.ds(0, EB)])
            pltpu.async_copy(hp_hbm.at[idx_v], rows_v, gsem).wait()

            def ebody(e, carry2):
                dwin = dst_v[pl.ds(e, 16)]
                fbase = jnp.zeros((16,), jnp.int32) + dwin[0] * D
                for j in range(D // 16):
                    vals = rows_v[e, pl.ds(16 * j, 16)]
                    plsc.addupdate_scatter(
                        acc_v, [fbase + (iota + 16 * j)], vals)
                return carry2

            lax.fori_loop(0, EB, ebody, 0)
            return carry

        lax.fori_loop(0, nb, body, 0)
        pltpu.sync_copy(acc_v.at[pl.ds(0, CH * D)],
                        out_hbm.at[pl.ds(rbase * D, CH * D)])


def _propagate(hp, src_pad, dstl_pad, desc):
    fn = pl.kernel(
        _sc_prop_body,
        out_type=jax.ShapeDtypeStruct((N_PAD * D,), jnp.float32),
        mesh=plsc.VectorSubcoreMesh(core_axis_name="c",
                                    subcore_axis_name="s"),
        compiler_params=pltpu.CompilerParams(needs_layout_passes=False),
        scratch_types=[
            pltpu.VMEM((16,), jnp.int32),
            pltpu.VMEM((EB,), jnp.int32),
            pltpu.VMEM((EB + 16,), jnp.int32),
            pltpu.VMEM((EB, D), jnp.float32),
            pltpu.VMEM(((CH + 8) * D,), jnp.float32),
            pltpu.SemaphoreType.DMA,
        ],
    )
    return fn(hp, hp.reshape(-1), src_pad, dstl_pad, desc).reshape(N_PAD, D)


def _edge_plan(dst_sorted, order, src):
    e = dst_sorted.shape[0]
    region = dst_sorted // CH
    r_starts = jnp.searchsorted(dst_sorted, jnp.arange(0, N_PAD + 1, CH))
    r_counts = jnp.diff(r_starts)
    local_rank = jnp.arange(e, dtype=jnp.int32) - r_starts[region]
    nb = (r_counts + EB - 1) // EB
    pad_starts = jnp.concatenate(
        [jnp.zeros((1,), nb.dtype), jnp.cumsum(nb * EB)])
    pos = pad_starts[region] + local_rank
    src_pad = jnp.zeros((EPAD,), jnp.int32).at[pos].set(src[order])
    dstl_pad = jnp.full((EPAD,), CH, jnp.int32).at[pos].set(
        dst_sorted - region * CH)
    batch_start = (pad_starts[:N_RT] // EB).astype(jnp.int32)
    desc = jnp.stack(
        [batch_start.reshape(NW, CHUNKS),
         nb.astype(jnp.int32).reshape(NW, CHUNKS)], axis=-1)
    desc = desc.reshape(NW, 2 * CHUNKS)
    desc = jnp.concatenate(
        [desc, jnp.zeros((NW, 16 - 2 * CHUNKS), jnp.int32)], axis=1)
    return src_pad, dstl_pad, desc


def kernel(x, edge_index, W0, b0, W1, b1, W2, b2):
    n = x.shape[0]
    src = edge_index[0].astype(jnp.int32)
    dst = edge_index[1].astype(jnp.int32)

    order = jnp.argsort(dst)
    dst_sorted = dst[order]
    offs = jnp.searchsorted(dst_sorted, jnp.arange(n + 1))
    deg = (offs[1:] - offs[:-1]).astype(jnp.float32) + 1.0
    dinv = jax.lax.rsqrt(deg)
    dinv = jnp.pad(dinv, (0, N_PAD - n)).reshape(N_PAD, 1)
    xp = jnp.pad(x, ((0, N_PAD - n), (0, 0)))
    src_pad, dstl_pad, desc = _edge_plan(dst_sorted, order, src)

    hp = _mm_a(xp, W0, dinv)
    g = _propagate(hp, src_pad, dstl_pad, desc)
    y0, hp = _mm_b(g, dinv, b0, W1[0])
    g = _propagate(hp, src_pad, dstl_pad, desc)
    _, hp = _mm_b(g, dinv, b1[0], W2[0])
    g = _propagate(hp, src_pad, dstl_pad, desc)
    y1, hp = _mm_br(g, dinv, b2[0], y0, W1[1])
    g = _propagate(hp, src_pad, dstl_pad, desc)
    _, hp = _mm_b(g, dinv, b1[1], W2[1])
    g = _propagate(hp, src_pad, dstl_pad, desc)
    y2, hp = _mm_br(g, dinv, b2[1], y1, W1[2])
    g = _propagate(hp, src_pad, dstl_pad, desc)
    _, hp = _mm_b(g, dinv, b1[2], W2[2])
    g = _propagate(hp, src_pad, dstl_pad, desc)
    y3 = _ew(g, dinv, b2[2], y2)

    return (y1[:n], y2[:n], y3[:n])


# speedup vs baseline: 1.8260x; 1.8260x over previous
import functools
import jax
import jax.numpy as jnp
from jax import lax
from jax.experimental import pallas as pl
from jax.experimental.pallas import tpu as pltpu
from jax.experimental.pallas import tpu_sc as plsc

N_PAD = 10240
ROW_BLK = 1024
D = 512
NC = 2
NS = 16
NW = NC * NS
CHUNKS = 2
N_RT = NW * CHUNKS
CH = N_PAD // N_RT
EB = 64
EPAD = 160000 + N_RT * EB


def _mm_a_body(x_ref, w_ref, dinv_ref, o_ref):
    o_ref[...] = jnp.dot(x_ref[...], w_ref[...],
                         preferred_element_type=jnp.float32) * dinv_ref[...]


def _mm_b_body(g_ref, dinv_ref, b_ref, w_ref, y_ref, h_ref):
    y = jnp.maximum(g_ref[...] * dinv_ref[...] + b_ref[...], 0.0)
    y_ref[...] = y
    h_ref[...] = jnp.dot(y, w_ref[...],
                         preferred_element_type=jnp.float32) * dinv_ref[...]


def _mm_br_body(g_ref, dinv_ref, b_ref, r_ref, w_ref, y_ref, h_ref):
    y = jnp.maximum(r_ref[...] + g_ref[...] * dinv_ref[...] + b_ref[...], 0.0)
    y_ref[...] = y
    h_ref[...] = jnp.dot(y, w_ref[...],
                         preferred_element_type=jnp.float32) * dinv_ref[...]


def _ew_body(g_ref, dinv_ref, b_ref, r_ref, y_ref):
    y_ref[...] = jnp.maximum(
        r_ref[...] + g_ref[...] * dinv_ref[...] + b_ref[...], 0.0)


def _row_spec(width):
    return pl.BlockSpec((ROW_BLK, width), lambda i: (i, 0))


def _full_spec(shape):
    return pl.BlockSpec(shape, lambda i: (0, 0))


def _mm_a(x, w, dinv):
    k = x.shape[1]
    n = w.shape[1]
    return pl.pallas_call(
        _mm_a_body,
        grid=(N_PAD // ROW_BLK,),
        in_specs=[_row_spec(k), _full_spec((k, n)), _row_spec(1)],
        out_specs=_row_spec(n),
        out_shape=jax.ShapeDtypeStruct((N_PAD, n), jnp.float32),
    )(x, w, dinv)


def _mm_b(g, dinv, b, w):
    n = w.shape[1]
    k = g.shape[1]
    return pl.pallas_call(
        _mm_b_body,
        grid=(N_PAD // ROW_BLK,),
        in_specs=[_row_spec(k), _row_spec(1), _full_spec((1, k)),
                  _full_spec((k, n))],
        out_specs=[_row_spec(k), _row_spec(n)],
        out_shape=[jax.ShapeDtypeStruct((N_PAD, k), jnp.float32),
                   jax.ShapeDtypeStruct((N_PAD, n), jnp.float32)],
    )(g, dinv, b.reshape(1, k), w)


def _mm_br(g, dinv, b, resid, w):
    n = w.shape[1]
    k = g.shape[1]
    return pl.pallas_call(
        _mm_br_body,
        grid=(N_PAD // ROW_BLK,),
        in_specs=[_row_spec(k), _row_spec(1), _full_spec((1, k)),
                  _row_spec(k), _full_spec((k, n))],
        out_specs=[_row_spec(k), _row_spec(n)],
        out_shape=[jax.ShapeDtypeStruct((N_PAD, k), jnp.float32),
                   jax.ShapeDtypeStruct((N_PAD, n), jnp.float32)],
    )(g, dinv, b.reshape(1, k), resid, w)


def _ew(g, dinv, b, resid):
    k = g.shape[1]
    return pl.pallas_call(
        _ew_body,
        grid=(N_PAD // ROW_BLK,),
        in_specs=[_row_spec(k), _row_spec(1), _full_spec((1, k)),
                  _row_spec(k)],
        out_specs=_row_spec(k),
        out_shape=jax.ShapeDtypeStruct((N_PAD, k), jnp.float32),
    )(g, dinv, b.reshape(1, k), resid)


def _sc_prop_body(hp_hbm, hp1d_hbm, srcp_hbm, dstl_hbm, desc_hbm, out_hbm,
                  desc_v, idx_v, dst_v, rows_v, acc_v, gsem):
    c = lax.axis_index("c")
    s = lax.axis_index("s")
    wid = c * NS + s
    pltpu.sync_copy(desc_hbm.at[wid], desc_v)
    dv = desc_v[...]
    iota = lax.iota(jnp.int32, 16)
    for k in range(CHUNKS):
        rbase = (wid * CHUNKS + k) * CH
        pltpu.sync_copy(hp1d_hbm.at[pl.ds(rbase * D, CH * D)],
                        acc_v.at[pl.ds(0, CH * D)])
        start = dv[2 * k]
        nb = dv[2 * k + 1]

        def body(i, carry):
            slot = pl.multiple_of((start + i) * EB, 8)
            pltpu.sync_copy(srcp_hbm.at[pl.ds(slot, EB)], idx_v)
            pltpu.sync_copy(dstl_hbm.at[pl.ds(slot, EB)],
                            dst_v.at[pl.ds(0, EB)])
            pltpu.async_copy(hp_hbm.at[idx_v], rows_v, gsem).wait()

            def ebody(e, carry2):
                base = e * 4
                dwin = dst_v[pl.ds(base, 16)]
                for u in range(4):
                    fbase = jnp.zeros((16,), jnp.int32) + dwin[u] * D
                    for j in range(D // 16):
                        vals = rows_v[base + u, pl.ds(16 * j, 16)]
                        plsc.addupdate_scatter(
                            acc_v, [fbase + (iota + 16 * j)], vals)
                return carry2

            lax.fori_loop(0, EB // 4, ebody, 0)
            return carry

        lax.fori_loop(0, nb, body, 0)
        pltpu.sync_copy(acc_v.at[pl.ds(0, CH * D)],
                        out_hbm.at[pl.ds(rbase * D, CH * D)])


def _propagate(hp, src_pad, dstl_pad, desc):
    fn = pl.kernel(
        _sc_prop_body,
        out_type=jax.ShapeDtypeStruct((N_PAD * D,), jnp.float32),
        mesh=plsc.VectorSubcoreMesh(core_axis_name="c",
                                    subcore_axis_name="s"),
        compiler_params=pltpu.CompilerParams(needs_layout_passes=False),
        scratch_types=[
            pltpu.VMEM((16,), jnp.int32),
            pltpu.VMEM((EB,), jnp.int32),
            pltpu.VMEM((EB + 16,), jnp.int32),
            pltpu.VMEM((EB, D), jnp.float32),
            pltpu.VMEM(((CH + 8) * D,), jnp.float32),
            pltpu.SemaphoreType.DMA,
        ],
    )
    return fn(hp, hp.reshape(-1), src_pad, dstl_pad, desc).reshape(N_PAD, D)


def _edge_plan(src, dst):
    region = dst // CH
    onehot = (region[:, None]
              == jnp.arange(N_RT, dtype=region.dtype)).astype(jnp.int32)
    pref = jnp.cumsum(onehot, axis=0)
    rank = jnp.take_along_axis(pref, region[:, None], axis=1)[:, 0] - 1
    r_counts = pref[-1]
    nb = (r_counts + EB - 1) // EB
    pad_starts = jnp.concatenate(
        [jnp.zeros((1,), nb.dtype), jnp.cumsum(nb * EB)])
    pos = pad_starts[region] + rank
    src_pad = jnp.zeros((EPAD,), jnp.int32).at[pos].set(src)
    dstl_pad = jnp.full((EPAD,), CH, jnp.int32).at[pos].set(
        dst - region * CH)
    batch_start = (pad_starts[:N_RT] // EB).astype(jnp.int32)
    desc = jnp.stack(
        [batch_start.reshape(NW, CHUNKS),
         nb.astype(jnp.int32).reshape(NW, CHUNKS)], axis=-1)
    desc = desc.reshape(NW, 2 * CHUNKS)
    desc = jnp.concatenate(
        [desc, jnp.zeros((NW, 16 - 2 * CHUNKS), jnp.int32)], axis=1)
    return src_pad, dstl_pad, desc


def kernel(x, edge_index, W0, b0, W1, b1, W2, b2):
    n = x.shape[0]
    src = edge_index[0].astype(jnp.int32)
    dst = edge_index[1].astype(jnp.int32)

    deg = jnp.zeros((n,), jnp.float32).at[dst].add(1.0) + 1.0
    dinv = jax.lax.rsqrt(deg)
    dinv = jnp.pad(dinv, (0, N_PAD - n)).reshape(N_PAD, 1)
    xp = jnp.pad(x, ((0, N_PAD - n), (0, 0)))
    src_pad, dstl_pad, desc = _edge_plan(src, dst)

    hp = _mm_a(xp, W0, dinv)
    g = _propagate(hp, src_pad, dstl_pad, desc)
    y0, hp = _mm_b(g, dinv, b0, W1[0])
    g = _propagate(hp, src_pad, dstl_pad, desc)
    _, hp = _mm_b(g, dinv, b1[0], W2[0])
    g = _propagate(hp, src_pad, dstl_pad, desc)
    y1, hp = _mm_br(g, dinv, b2[0], y0, W1[1])
    g = _propagate(hp, src_pad, dstl_pad, desc)
    _, hp = _mm_b(g, dinv, b1[1], W2[1])
    g = _propagate(hp, src_pad, dstl_pad, desc)
    y2, hp = _mm_br(g, dinv, b2[1], y1, W1[2])
    g = _propagate(hp, src_pad, dstl_pad, desc)
    _, hp = _mm_b(g, dinv, b1[2], W2[2])
    g = _propagate(hp, src_pad, dstl_pad, desc)
    y3 = _ew(g, dinv, b2[2], y2)

    return (y1[:n], y2[:n], y3[:n])


# speedup vs baseline: 1.8742x; 1.0264x over previous
import functools
import jax
import jax.numpy as jnp
from jax import lax
from jax.experimental import pallas as pl
from jax.experimental.pallas import tpu as pltpu
from jax.experimental.pallas import tpu_sc as plsc

N_PAD = 10240
ROW_BLK = 1024
D = 512
NC = 2
NS = 16
NW = NC * NS
CHUNKS = 2
N_RT = NW * CHUNKS
CH = N_PAD // N_RT
EB = 32
EPAD = 160000 + N_RT * 2 * EB


def _mm_a_body(x_ref, w_ref, dinv_ref, o_ref):
    o_ref[...] = jnp.dot(x_ref[...], w_ref[...],
                         preferred_element_type=jnp.float32) * dinv_ref[...]


def _mm_b_body(g_ref, dinv_ref, b_ref, w_ref, y_ref, h_ref):
    y = jnp.maximum(g_ref[...] * dinv_ref[...] + b_ref[...], 0.0)
    y_ref[...] = y
    h_ref[...] = jnp.dot(y, w_ref[...],
                         preferred_element_type=jnp.float32) * dinv_ref[...]


def _mm_br_body(g_ref, dinv_ref, b_ref, r_ref, w_ref, y_ref, h_ref):
    y = jnp.maximum(r_ref[...] + g_ref[...] * dinv_ref[...] + b_ref[...], 0.0)
    y_ref[...] = y
    h_ref[...] = jnp.dot(y, w_ref[...],
                         preferred_element_type=jnp.float32) * dinv_ref[...]


def _ew_body(g_ref, dinv_ref, b_ref, r_ref, y_ref):
    y_ref[...] = jnp.maximum(
        r_ref[...] + g_ref[...] * dinv_ref[...] + b_ref[...], 0.0)


def _row_spec(width):
    return pl.BlockSpec((ROW_BLK, width), lambda i: (i, 0))


def _full_spec(shape):
    return pl.BlockSpec(shape, lambda i: (0, 0))


def _mm_a(x, w, dinv):
    k = x.shape[1]
    n = w.shape[1]
    return pl.pallas_call(
        _mm_a_body,
        grid=(N_PAD // ROW_BLK,),
        in_specs=[_row_spec(k), _full_spec((k, n)), _row_spec(1)],
        out_specs=_row_spec(n),
        out_shape=jax.ShapeDtypeStruct((N_PAD, n), jnp.float32),
    )(x, w, dinv)


def _mm_b(g, dinv, b, w):
    n = w.shape[1]
    k = g.shape[1]
    return pl.pallas_call(
        _mm_b_body,
        grid=(N_PAD // ROW_BLK,),
        in_specs=[_row_spec(k), _row_spec(1), _full_spec((1, k)),
                  _full_spec((k, n))],
        out_specs=[_row_spec(k), _row_spec(n)],
        out_shape=[jax.ShapeDtypeStruct((N_PAD, k), jnp.float32),
                   jax.ShapeDtypeStruct((N_PAD, n), jnp.float32)],
    )(g, dinv, b.reshape(1, k), w)


def _mm_br(g, dinv, b, resid, w):
    n = w.shape[1]
    k = g.shape[1]
    return pl.pallas_call(
        _mm_br_body,
        grid=(N_PAD // ROW_BLK,),
        in_specs=[_row_spec(k), _row_spec(1), _full_spec((1, k)),
                  _row_spec(k), _full_spec((k, n))],
        out_specs=[_row_spec(k), _row_spec(n)],
        out_shape=[jax.ShapeDtypeStruct((N_PAD, k), jnp.float32),
                   jax.ShapeDtypeStruct((N_PAD, n), jnp.float32)],
    )(g, dinv, b.reshape(1, k), resid, w)


def _ew(g, dinv, b, resid):
    k = g.shape[1]
    return pl.pallas_call(
        _ew_body,
        grid=(N_PAD // ROW_BLK,),
        in_specs=[_row_spec(k), _row_spec(1), _full_spec((1, k)),
                  _row_spec(k)],
        out_specs=_row_spec(k),
        out_shape=jax.ShapeDtypeStruct((N_PAD, k), jnp.float32),
    )(g, dinv, b.reshape(1, k), resid)


def _sc_prop_body(hp_hbm, hp1d_hbm, srcp_hbm, dstl_hbm, desc_hbm, out_hbm,
                  desc_v, idx_a, idx_b, dst_a, dst_b, rows_a, rows_b,
                  acc_v, sem_a, sem_b):
    c = lax.axis_index("c")
    s = lax.axis_index("s")
    wid = c * NS + s
    pltpu.sync_copy(desc_hbm.at[wid], desc_v)
    dv = desc_v[...]
    iota = lax.iota(jnp.int32, 16)

    def accum(rows_v, dst_v):
        def ebody(e, carry2):
            base = e * 4
            dwin = dst_v[pl.ds(base, 16)]
            for u in range(4):
                fbase = jnp.zeros((16,), jnp.int32) + dwin[u] * D
                for j in range(D // 16):
                    vals = rows_v[base + u, pl.ds(16 * j, 16)]
                    plsc.addupdate_scatter(
                        acc_v, [fbase + (iota + 16 * j)], vals)
            return carry2

        lax.fori_loop(0, EB // 4, ebody, 0)

    for k in range(CHUNKS):
        rbase = (wid * CHUNKS + k) * CH
        pltpu.sync_copy(hp1d_hbm.at[pl.ds(rbase * D, CH * D)],
                        acc_v.at[pl.ds(0, CH * D)])
        start = dv[2 * k]
        nb2 = dv[2 * k + 1]

        def body(i, carry):
            slot0 = pl.multiple_of((start + 2 * i) * EB, 8)
            slot1 = pl.multiple_of((start + 2 * i + 1) * EB, 8)
            pltpu.sync_copy(srcp_hbm.at[pl.ds(slot0, EB)], idx_a)
            cp_a = pltpu.async_copy(hp_hbm.at[idx_a], rows_a, sem_a)
            pltpu.sync_copy(srcp_hbm.at[pl.ds(slot1, EB)], idx_b)
            cp_b = pltpu.async_copy(hp_hbm.at[idx_b], rows_b, sem_b)
            pltpu.sync_copy(dstl_hbm.at[pl.ds(slot0, EB)],
                            dst_a.at[pl.ds(0, EB)])
            cp_a.wait()
            accum(rows_a, dst_a)
            pltpu.sync_copy(dstl_hbm.at[pl.ds(slot1, EB)],
                            dst_b.at[pl.ds(0, EB)])
            cp_b.wait()
            accum(rows_b, dst_b)
            return carry

        lax.fori_loop(0, nb2, body, 0)
        pltpu.sync_copy(acc_v.at[pl.ds(0, CH * D)],
                        out_hbm.at[pl.ds(rbase * D, CH * D)])


def _propagate(hp, src_pad, dstl_pad, desc):
    fn = pl.kernel(
        _sc_prop_body,
        out_type=jax.ShapeDtypeStruct((N_PAD * D,), jnp.float32),
        mesh=plsc.VectorSubcoreMesh(core_axis_name="c",
                                    subcore_axis_name="s"),
        compiler_params=pltpu.CompilerParams(needs_layout_passes=False),
        scratch_types=[
            pltpu.VMEM((16,), jnp.int32),
            pltpu.VMEM((EB,), jnp.int32),
            pltpu.VMEM((EB,), jnp.int32),
            pltpu.VMEM((EB + 16,), jnp.int32),
            pltpu.VMEM((EB + 16,), jnp.int32),
            pltpu.VMEM((EB, D), jnp.float32),
            pltpu.VMEM((EB, D), jnp.float32),
            pltpu.VMEM(((CH + 8) * D,), jnp.float32),
            pltpu.SemaphoreType.DMA,
            pltpu.SemaphoreType.DMA,
        ],
    )
    return fn(hp, hp.reshape(-1), src_pad, dstl_pad, desc).reshape(N_PAD, D)


def _edge_plan(src, dst):
    region = dst // CH
    onehot = (region[:, None]
              == jnp.arange(N_RT, dtype=region.dtype)).astype(jnp.int32)
    pref = jnp.cumsum(onehot, axis=0)
    rank = jnp.take_along_axis(pref, region[:, None], axis=1)[:, 0] - 1
    r_counts = pref[-1]
    nb = (r_counts + 2 * EB - 1) // (2 * EB)
    pad_starts = jnp.concatenate(
        [jnp.zeros((1,), nb.dtype), jnp.cumsum(nb * 2 * EB)])
    pos = pad_starts[region] + rank
    src_pad = jnp.zeros((EPAD,), jnp.int32).at[pos].set(src)
    dstl_pad = jnp.full((EPAD,), CH, jnp.int32).at[pos].set(
        dst - region * CH)
    batch_start = (pad_starts[:N_RT] // EB).astype(jnp.int32)
    desc = jnp.stack(
        [batch_start.reshape(NW, CHUNKS),
         nb.astype(jnp.int32).reshape(NW, CHUNKS)], axis=-1)
    desc = desc.reshape(NW, 2 * CHUNKS)
    desc = jnp.concatenate(
        [desc, jnp.zeros((NW, 16 - 2 * CHUNKS), jnp.int32)], axis=1)
    return src_pad, dstl_pad, desc


def kernel(x, edge_index, W0, b0, W1, b1, W2, b2):
    n = x.shape[0]
    src = edge_index[0].astype(jnp.int32)
    dst = edge_index[1].astype(jnp.int32)

    deg = jnp.zeros((n,), jnp.float32).at[dst].add(1.0) + 1.0
    dinv = jax.lax.rsqrt(deg)
    dinv = jnp.pad(dinv, (0, N_PAD - n)).reshape(N_PAD, 1)
    xp = jnp.pad(x, ((0, N_PAD - n), (0, 0)))
    src_pad, dstl_pad, desc = _edge_plan(src, dst)

    hp = _mm_a(xp, W0, dinv)
    g = _propagate(hp, src_pad, dstl_pad, desc)
    y0, hp = _mm_b(g, dinv, b0, W1[0])
    g = _propagate(hp, src_pad, dstl_pad, desc)
    _, hp = _mm_b(g, dinv, b1[0], W2[0])
    g = _propagate(hp, src_pad, dstl_pad, desc)
    y1, hp = _mm_br(g, dinv, b2[0], y0, W1[1])
    g = _propagate(hp, src_pad, dstl_pad, desc)
    _, hp = _mm_b(g, dinv, b1[1], W2[1])
    g = _propagate(hp, src_pad, dstl_pad, desc)
    y2, hp = _mm_br(g, dinv, b2[1], y1, W1[2])
    g = _propagate(hp, src_pad, dstl_pad, desc)
    _, hp = _mm_b(g, dinv, b1[2], W2[2])
    g = _propagate(hp, src_pad, dstl_pad, desc)
    y3 = _ew(g, dinv, b2[2], y2)

    return (y1[:n], y2[:n], y3[:n])


# speedup vs baseline: 2.1837x; 1.1652x over previous
import functools
import jax
import jax.numpy as jnp
from jax import lax
from jax.experimental import pallas as pl
from jax.experimental.pallas import tpu as pltpu
from jax.experimental.pallas import tpu_sc as plsc

N_PAD = 10240
ROW_BLK = 1024
D = 512
NC = 2
NS = 16
NW = NC * NS
CHUNKS = 2
N_RT = NW * CHUNKS
CH = N_PAD // N_RT
EB = 32
EPAD = 160000 + N_RT * 2 * EB


def _mm_a_body(x_ref, w_ref, dinv_ref, o_ref):
    o_ref[...] = jnp.dot(x_ref[...], w_ref[...],
                         preferred_element_type=jnp.float32) * dinv_ref[...]


def _mm_b_body(g_ref, dinv_ref, b_ref, w_ref, y_ref, h_ref):
    y = jnp.maximum(g_ref[...] * dinv_ref[...] + b_ref[...], 0.0)
    y_ref[...] = y
    h_ref[...] = jnp.dot(y, w_ref[...],
                         preferred_element_type=jnp.float32) * dinv_ref[...]


def _mm_br_body(g_ref, dinv_ref, b_ref, r_ref, w_ref, y_ref, h_ref):
    y = jnp.maximum(r_ref[...] + g_ref[...] * dinv_ref[...] + b_ref[...], 0.0)
    y_ref[...] = y
    h_ref[...] = jnp.dot(y, w_ref[...],
                         preferred_element_type=jnp.float32) * dinv_ref[...]


def _ew_body(g_ref, dinv_ref, b_ref, r_ref, y_ref):
    y_ref[...] = jnp.maximum(
        r_ref[...] + g_ref[...] * dinv_ref[...] + b_ref[...], 0.0)


def _row_spec(width):
    return pl.BlockSpec((ROW_BLK, width), lambda i: (i, 0))


def _full_spec(shape):
    return pl.BlockSpec(shape, lambda i: (0, 0))


def _mm_a(x, w, dinv):
    k = x.shape[1]
    n = w.shape[1]
    return pl.pallas_call(
        _mm_a_body,
        grid=(N_PAD // ROW_BLK,),
        in_specs=[_row_spec(k), _full_spec((k, n)), _row_spec(1)],
        out_specs=_row_spec(n),
        out_shape=jax.ShapeDtypeStruct((N_PAD, n), jnp.float32),
    )(x, w, dinv)


def _mm_b(g, dinv, b, w):
    n = w.shape[1]
    k = g.shape[1]
    return pl.pallas_call(
        _mm_b_body,
        grid=(N_PAD // ROW_BLK,),
        in_specs=[_row_spec(k), _row_spec(1), _full_spec((1, k)),
                  _full_spec((k, n))],
        out_specs=[_row_spec(k), _row_spec(n)],
        out_shape=[jax.ShapeDtypeStruct((N_PAD, k), jnp.float32),
                   jax.ShapeDtypeStruct((N_PAD, n), jnp.float32)],
    )(g, dinv, b.reshape(1, k), w)


def _mm_br(g, dinv, b, resid, w):
    n = w.shape[1]
    k = g.shape[1]
    return pl.pallas_call(
        _mm_br_body,
        grid=(N_PAD // ROW_BLK,),
        in_specs=[_row_spec(k), _row_spec(1), _full_spec((1, k)),
                  _row_spec(k), _full_spec((k, n))],
        out_specs=[_row_spec(k), _row_spec(n)],
        out_shape=[jax.ShapeDtypeStruct((N_PAD, k), jnp.float32),
                   jax.ShapeDtypeStruct((N_PAD, n), jnp.float32)],
    )(g, dinv, b.reshape(1, k), resid, w)


def _ew(g, dinv, b, resid):
    k = g.shape[1]
    return pl.pallas_call(
        _ew_body,
        grid=(N_PAD // ROW_BLK,),
        in_specs=[_row_spec(k), _row_spec(1), _full_spec((1, k)),
                  _row_spec(k)],
        out_specs=_row_spec(k),
        out_shape=jax.ShapeDtypeStruct((N_PAD, k), jnp.float32),
    )(g, dinv, b.reshape(1, k), resid)


def _sc_prop_body(hp_hbm, hp1d_hbm, srcp_hbm, dstl_hbm, desc_hbm, out_hbm,
                  desc_v, idx_a, idx_b, dst_a, dst_b, rows_a, rows_b,
                  acc_v, sem_a, sem_b):
    c = lax.axis_index("c")
    s = lax.axis_index("s")
    wid = c * NS + s
    pltpu.sync_copy(desc_hbm.at[wid], desc_v)
    dv = desc_v[...]
    iota = lax.iota(jnp.int32, 16)

    def accum(rows_v, dst_v):
        @plsc.parallel_loop(0, EB // 4, step=1, carry=jnp.int32(0))
        def ebody(e, carry2):
            base = e * 4
            dwin = dst_v[pl.ds(base, 16)]
            for u in range(4):
                fbase = jnp.zeros((16,), jnp.int32) + dwin[u] * D
                for j in range(D // 16):
                    vals = rows_v[base + u, pl.ds(16 * j, 16)]
                    plsc.addupdate_scatter(
                        acc_v, [fbase + (iota + 16 * j)], vals)
            return carry2

    for k in range(CHUNKS):
        rbase = (wid * CHUNKS + k) * CH
        pltpu.sync_copy(hp1d_hbm.at[pl.ds(rbase * D, CH * D)],
                        acc_v.at[pl.ds(0, CH * D)])
        start = dv[2 * k]
        nb2 = dv[2 * k + 1]

        def body(i, carry):
            slot0 = pl.multiple_of((start + 2 * i) * EB, 8)
            slot1 = pl.multiple_of((start + 2 * i + 1) * EB, 8)
            pltpu.sync_copy(srcp_hbm.at[pl.ds(slot0, EB)], idx_a)
            cp_a = pltpu.async_copy(hp_hbm.at[idx_a], rows_a, sem_a)
            pltpu.sync_copy(srcp_hbm.at[pl.ds(slot1, EB)], idx_b)
            cp_b = pltpu.async_copy(hp_hbm.at[idx_b], rows_b, sem_b)
            pltpu.sync_copy(dstl_hbm.at[pl.ds(slot0, EB)],
                            dst_a.at[pl.ds(0, EB)])
            cp_a.wait()
            accum(rows_a, dst_a)
            pltpu.sync_copy(dstl_hbm.at[pl.ds(slot1, EB)],
                            dst_b.at[pl.ds(0, EB)])
            cp_b.wait()
            accum(rows_b, dst_b)
            return carry

        lax.fori_loop(0, nb2, body, 0)
        pltpu.sync_copy(acc_v.at[pl.ds(0, CH * D)],
                        out_hbm.at[pl.ds(rbase * D, CH * D)])


def _propagate(hp, src_pad, dstl_pad, desc):
    fn = pl.kernel(
        _sc_prop_body,
        out_type=jax.ShapeDtypeStruct((N_PAD * D,), jnp.float32),
        mesh=plsc.VectorSubcoreMesh(core_axis_name="c",
                                    subcore_axis_name="s"),
        compiler_params=pltpu.CompilerParams(needs_layout_passes=False),
        scratch_types=[
            pltpu.VMEM((16,), jnp.int32),
            pltpu.VMEM((EB,), jnp.int32),
            pltpu.VMEM((EB,), jnp.int32),
            pltpu.VMEM((EB + 16,), jnp.int32),
            pltpu.VMEM((EB + 16,), jnp.int32),
            pltpu.VMEM((EB, D), jnp.float32),
            pltpu.VMEM((EB, D), jnp.float32),
            pltpu.VMEM(((CH + 8) * D,), jnp.float32),
            pltpu.SemaphoreType.DMA,
            pltpu.SemaphoreType.DMA,
        ],
    )
    return fn(hp, hp.reshape(-1), src_pad, dstl_pad, desc).reshape(N_PAD, D)


def _edge_plan(src, dst):
    region = dst // CH
    onehot = (region[:, None]
              == jnp.arange(N_RT, dtype=region.dtype)).astype(jnp.int32)
    pref = jnp.cumsum(onehot, axis=0)
    rank = jnp.take_along_axis(pref, region[:, None], axis=1)[:, 0] - 1
    r_counts = pref[-1]
    nb = (r_counts + 2 * EB - 1) // (2 * EB)
    pad_starts = jnp.concatenate(
        [jnp.zeros((1,), nb.dtype), jnp.cumsum(nb * 2 * EB)])
    pos = pad_starts[region] + rank
    src_pad = jnp.zeros((EPAD,), jnp.int32).at[pos].set(src)
    dstl_pad = jnp.full((EPAD,), CH, jnp.int32).at[pos].set(
        dst - region * CH)
    batch_start = (pad_starts[:N_RT] // EB).astype(jnp.int32)
    desc = jnp.stack(
        [batch_start.reshape(NW, CHUNKS),
         nb.astype(jnp.int32).reshape(NW, CHUNKS)], axis=-1)
    desc = desc.reshape(NW, 2 * CHUNKS)
    desc = jnp.concatenate(
        [desc, jnp.zeros((NW, 16 - 2 * CHUNKS), jnp.int32)], axis=1)
    return src_pad, dstl_pad, desc


def kernel(x, edge_index, W0, b0, W1, b1, W2, b2):
    n = x.shape[0]
    src = edge_index[0].astype(jnp.int32)
    dst = edge_index[1].astype(jnp.int32)

    deg = jnp.zeros((n,), jnp.float32).at[dst].add(1.0) + 1.0
    dinv = jax.lax.rsqrt(deg)
    dinv = jnp.pad(dinv, (0, N_PAD - n)).reshape(N_PAD, 1)
    xp = jnp.pad(x, ((0, N_PAD - n), (0, 0)))
    src_pad, dstl_pad, desc = _edge_plan(src, dst)

    hp = _mm_a(xp, W0, dinv)
    g = _propagate(hp, src_pad, dstl_pad, desc)
    y0, hp = _mm_b(g, dinv, b0, W1[0])
    g = _propagate(hp, src_pad, dstl_pad, desc)
    _, hp = _mm_b(g, dinv, b1[0], W2[0])
    g = _propagate(hp, src_pad, dstl_pad, desc)
    y1, hp = _mm_br(g, dinv, b2[0], y0, W1[1])
    g = _propagate(hp, src_pad, dstl_pad, desc)
    _, hp = _mm_b(g, dinv, b1[1], W2[1])
    g = _propagate(hp, src_pad, dstl_pad, desc)
    y2, hp = _mm_br(g, dinv, b2[1], y1, W1[2])
    g = _propagate(hp, src_pad, dstl_pad, desc)
    _, hp = _mm_b(g, dinv, b1[2], W2[2])
    g = _propagate(hp, src_pad, dstl_pad, desc)
    y3 = _ew(g, dinv, b2[2], y2)

    return (y1[:n], y2[:n], y3[:n])
